# baseline (speedup 1.0000x reference)
import functools

import jax
import jax.numpy as jnp
from jax import lax
from jax.experimental import pallas as pl
from jax.experimental.pallas import tpu as pltpu

_SUBLANE = 8
_BLOCK_BYTES = 4 * 1024 * 1024


def _round_up(n, m):
    return (n + m - 1) // m * m


def _gate_kernel(x_ref, w1_ref, b1_ref, w2_ref, b2_ref, o_ref, *, hw_true):
    x = x_ref[...]
    nb, hwp, c = x.shape
    if hwp != hw_true:
        row = lax.broadcasted_iota(jnp.int32, (nb, hwp, c), 1)
        valid = row < hw_true
        x_for_sum = jnp.where(valid, x, 0.0)
        x_for_max = jnp.where(valid, x, -jnp.inf)
    else:
        x_for_sum = x
        x_for_max = x

    avg = jnp.sum(x_for_sum, axis=1) * (1.0 / hw_true)
    mx = jnp.max(x_for_max, axis=1)

    w1 = w1_ref[...]
    b1 = b1_ref[...]
    w2 = w2_ref[...]
    b2 = b2_ref[...]

    dn = (((1,), (0,)), ((), ()))
    dot = functools.partial(lax.dot_general, dimension_numbers=dn,
                            preferred_element_type=jnp.float32,
                            precision=lax.Precision.HIGHEST)
    h_a = jnp.maximum(dot(avg, w1) + b1, 0.0)
    h_m = jnp.maximum(dot(mx, w1) + b1, 0.0)
    att = dot(h_a, w2) + dot(h_m, w2) + 2.0 * b2
    scale = jax.nn.sigmoid(att)
    o_ref[...] = x * scale[:, None, :]


def kernel(x, w1, b1, w2, b2):
    B, C, H, W = x.shape
    HW = H * W
    Ch = w1.shape[1]

    x_nhwc = jnp.transpose(x, (0, 2, 3, 1)).reshape(B, HW, C).astype(jnp.float32)
    w1 = w1.astype(jnp.float32)
    b1 = b1.astype(jnp.float32).reshape(1, Ch)
    w2 = w2.astype(jnp.float32)
    b2 = b2.astype(jnp.float32).reshape(1, C)

    HWp = _round_up(HW, _SUBLANE)
    if HWp != HW:
        x_nhwc = jnp.pad(x_nhwc, ((0, 0), (0, HWp - HW), (0, 0)))

    nb = 1
    for cand in (4, 2, 1):
        if B % cand == 0 and cand * C * HWp * 4 <= _BLOCK_BYTES:
            nb = cand
            break

    out = pl.pallas_call(
        functools.partial(_gate_kernel, hw_true=HW),
        out_shape=jax.ShapeDtypeStruct((B, HWp, C), jnp.float32),
        grid=(B // nb,),
        in_specs=[
            pl.BlockSpec((nb, HWp, C), lambda b: (b, 0, 0)),
            pl.BlockSpec((C, Ch), lambda b: (0, 0)),
            pl.BlockSpec((1, Ch), lambda b: (0, 0)),
            pl.BlockSpec((Ch, C), lambda b: (0, 0)),
            pl.BlockSpec((1, C), lambda b: (0, 0)),
        ],
        out_specs=pl.BlockSpec((nb, HWp, C), lambda b: (b, 0, 0)),
        compiler_params=pltpu.CompilerParams(
            dimension_semantics=("parallel",),
            vmem_limit_bytes=48 * 1024 * 1024,
        ),
    )(x_nhwc, w1, b1, w2, b2)

    if HWp != HW:
        out = out[:, :HW, :]
    return jnp.transpose(out.reshape(B, H, W, C), (0, 3, 1, 2))


# speedup vs baseline: 1.0268x; 1.0268x over previous
import functools

import jax
import jax.numpy as jnp
from jax import lax
from jax.experimental import pallas as pl
from jax.experimental.pallas import tpu as pltpu

_SUBLANE = 8
_BLOCK_BYTES = 8 * 1024 * 1024


def _round_up(n, m):
    return (n + m - 1) // m * m


def _gate_kernel(x_ref, w1_ref, b1_ref, w2_ref, b2_ref, o_ref, *, hw_true):
    x = x_ref[...]
    nb, hwp, c = x.shape
    if hwp != hw_true:
        row = lax.broadcasted_iota(jnp.int32, (nb, hwp, c), 1)
        valid = row < hw_true
        x_for_sum = jnp.where(valid, x, 0.0)
        x_for_max = jnp.where(valid, x, -jnp.inf)
    else:
        x_for_sum = x
        x_for_max = x

    avg = jnp.sum(x_for_sum, axis=1) * (1.0 / hw_true)
    mx = jnp.max(x_for_max, axis=1)

    w1 = w1_ref[...]
    b1 = b1_ref[...]
    w2 = w2_ref[...]
    b2 = b2_ref[...]

    dn = (((1,), (0,)), ((), ()))
    dot = functools.partial(lax.dot_general, dimension_numbers=dn,
                            preferred_element_type=jnp.float32,
                            precision=lax.Precision.HIGHEST)
    h_a = jnp.maximum(dot(avg, w1) + b1, 0.0)
    h_m = jnp.maximum(dot(mx, w1) + b1, 0.0)
    att = dot(h_a, w2) + dot(h_m, w2) + 2.0 * b2
    scale = jax.nn.sigmoid(att)
    o_ref[...] = x * scale[:, None, :]


def kernel(x, w1, b1, w2, b2):
    B, C, H, W = x.shape
    HW = H * W
    Ch = w1.shape[1]

    x_nhwc = jnp.transpose(x, (0, 2, 3, 1)).reshape(B, HW, C).astype(jnp.float32)
    w1 = w1.astype(jnp.float32)
    b1 = b1.astype(jnp.float32).reshape(1, Ch)
    w2 = w2.astype(jnp.float32)
    b2 = b2.astype(jnp.float32).reshape(1, C)

    HWp = _round_up(HW, _SUBLANE)
    if HWp != HW:
        x_nhwc = jnp.pad(x_nhwc, ((0, 0), (0, HWp - HW), (0, 0)))

    nb = 1
    for cand in (8, 4, 2, 1):
        if B % cand == 0 and cand * C * HWp * 4 <= _BLOCK_BYTES:
            nb = cand
            break

    out = pl.pallas_call(
        functools.partial(_gate_kernel, hw_true=HW),
        out_shape=jax.ShapeDtypeStruct((B, HWp, C), jnp.float32),
        grid=(B // nb,),
        in_specs=[
            pl.BlockSpec((nb, HWp, C), lambda b: (b, 0, 0)),
            pl.BlockSpec((C, Ch), lambda b: (0, 0)),
            pl.BlockSpec((1, Ch), lambda b: (0, 0)),
            pl.BlockSpec((Ch, C), lambda b: (0, 0)),
            pl.BlockSpec((1, C), lambda b: (0, 0)),
        ],
        out_specs=pl.BlockSpec((nb, HWp, C), lambda b: (b, 0, 0)),
        compiler_params=pltpu.CompilerParams(
            dimension_semantics=("parallel",),
            vmem_limit_bytes=48 * 1024 * 1024,
        ),
    )(x_nhwc, w1, b1, w2, b2)

    if HWp != HW:
        out = out[:, :HW, :]
    return jnp.transpose(out.reshape(B, H, W, C), (0, 3, 1, 2))


# speedup vs baseline: 1.0400x; 1.0128x over previous
import functools

import jax
import jax.numpy as jnp
from jax import lax
from jax.experimental import pallas as pl
from jax.experimental.pallas import tpu as pltpu

_SUBLANE = 8
_BLOCK_BYTES = 8 * 1024 * 1024


def _round_up(n, m):
    return (n + m - 1) // m * m


def _gate_kernel(x_ref, w1t_ref, b1_ref, w2_ref, b2_ref, o_ref, *, hw_true):
    x = x_ref[...]
    nb, hwp, c = x.shape
    if hwp != hw_true:
        row = lax.broadcasted_iota(jnp.int32, (nb, hwp, c), 1)
        valid = row < hw_true
        x_for_sum = jnp.where(valid, x, 0.0)
        x_for_max = jnp.where(valid, x, -jnp.inf)
    else:
        x_for_sum = x
        x_for_max = x

    avg = jnp.sum(x_for_sum, axis=1) * (1.0 / hw_true)
    mx = jnp.max(x_for_max, axis=1)

    w1t = w1t_ref[...]
    b1 = b1_ref[...]
    w2 = w2_ref[...]
    b2 = b2_ref[...]

    dot_t = functools.partial(lax.dot_general,
                              dimension_numbers=(((1,), (1,)), ((), ())),
                              preferred_element_type=jnp.float32,
                              precision=lax.Precision.HIGHEST)
    dot = functools.partial(lax.dot_general,
                            dimension_numbers=(((1,), (0,)), ((), ())),
                            preferred_element_type=jnp.float32,
                            precision=lax.Precision.HIGHEST)
    h_a = jnp.maximum(dot_t(avg, w1t) + b1, 0.0)
    h_m = jnp.maximum(dot_t(mx, w1t) + b1, 0.0)
    att = dot(h_a, w2) + dot(h_m, w2) + 2.0 * b2
    scale = jax.nn.sigmoid(att)
    o_ref[...] = x * scale[:, None, :]


def kernel(x, w1, b1, w2, b2):
    B, C, H, W = x.shape
    HW = H * W
    Ch = w1.shape[1]

    x_nhwc = jnp.transpose(x, (0, 2, 3, 1)).reshape(B, HW, C).astype(jnp.float32)
    w1t = jnp.transpose(w1.astype(jnp.float32))
    b1 = b1.astype(jnp.float32).reshape(1, Ch)
    w2 = w2.astype(jnp.float32)
    b2 = b2.astype(jnp.float32).reshape(1, C)

    HWp = _round_up(HW, _SUBLANE)
    if HWp != HW:
        x_nhwc = jnp.pad(x_nhwc, ((0, 0), (0, HWp - HW), (0, 0)))

    nb = 1
    for cand in (8, 4, 2, 1):
        if B % cand == 0 and cand * C * HWp * 4 <= _BLOCK_BYTES:
            nb = cand
            break

    out = pl.pallas_call(
        functools.partial(_gate_kernel, hw_true=HW),
        out_shape=jax.ShapeDtypeStruct((B, HWp, C), jnp.float32),
        grid=(B // nb,),
        in_specs=[
            pl.BlockSpec((nb, HWp, C), lambda b: (b, 0, 0)),
            pl.BlockSpec((Ch, C), lambda b: (0, 0)),
            pl.BlockSpec((1, Ch), lambda b: (0, 0)),
            pl.BlockSpec((Ch, C), lambda b: (0, 0)),
            pl.BlockSpec((1, C), lambda b: (0, 0)),
        ],
        out_specs=pl.BlockSpec((nb, HWp, C), lambda b: (b, 0, 0)),
        compiler_params=pltpu.CompilerParams(
            dimension_semantics=("parallel",),
            vmem_limit_bytes=48 * 1024 * 1024,
        ),
    )(x_nhwc, w1t, b1, w2, b2)

    if HWp != HW:
        out = out[:, :HW, :]
    return jnp.transpose(out.reshape(B, H, W, C), (0, 3, 1, 2))
